# baseline (speedup 1.0000x reference)
import functools

import jax
import jax.numpy as jnp
from jax import lax
from jax.experimental import pallas as pl
from jax.experimental.pallas import tpu as pltpu
from jax.experimental.pallas import tpu_sc as plsc

_N = 10000
_D = 128
_H = 88
_G = 64
_OUT = 100

_NP = 10240
_HP = 128
_CH = 128
_BLK = 8
_NB = 2
_R = 512


def _seg_sum_sc(y, src2d, dst2d):
    cpt = src2d.shape[0] // 32
    rpt = _NP // 16
    mesh = plsc.VectorSubcoreMesh(core_axis_name="c", subcore_axis_name="s")

    @functools.partial(
        pl.kernel,
        out_type=jax.ShapeDtypeStruct((2 * _NP, _HP), jnp.float32),
        mesh=mesh,
        scratch_types=[
            pltpu.VMEM((_BLK, _CH), jnp.int32),
            pltpu.VMEM((_BLK, _CH), jnp.int32),
            pltpu.VMEM((_NB, _CH, _HP), jnp.float32),
            pltpu.VMEM_SHARED((_NP, _HP), jnp.float32),
            pltpu.SemaphoreType.DMA,
            pltpu.SemaphoreType.DMA,
        ],
    )
    def k(y_hbm, src_hbm, dst_hbm, out_hbm, srcb, dstb, rows, acc,
          sem0, sem1):
        cid = lax.axis_index("c")
        sid = lax.axis_index("s")
        sems = (sem0, sem1)

        def zfill(i, carry):
            for kk in range(_HP // 16):
                rows[0, i, pl.ds(kk * 16, 16)] = jnp.zeros((16,), jnp.float32)
            return carry

        lax.fori_loop(0, _CH, zfill, 0)

        def zcopy(t, carry):
            pltpu.sync_copy(rows.at[0],
                            acc.at[pl.ds(sid * rpt + t * _CH, _CH)])
            return carry

        lax.fori_loop(0, rpt // _CH, zcopy, 0)
        plsc.subcore_barrier()

        def gcopy(jj, b):
            return pltpu.make_async_copy(y_hbm.at[srcb.at[jj]], rows.at[b],
                                         sems[b])

        def blk(bi, carry):
            base = (cid * 16 + sid) * cpt + bi * _BLK
            pltpu.sync_copy(src_hbm.at[pl.ds(base, _BLK)], srcb)
            pltpu.sync_copy(dst_hbm.at[pl.ds(base, _BLK)], dstb)
            for b in range(_NB):
                gcopy(b, b).start()

            def quad(p, c2):
                for b in range(_NB):
                    jj = p * _NB + b
                    gcopy(jj, b).wait()
                    pltpu.sync_copy(rows.at[b], acc.at[dstb.at[jj]],
                                    add=True)

                    @pl.when(jj + _NB < _BLK)
                    def _():
                        gcopy(jj + _NB, b).start()
                return c2

            lax.fori_loop(0, _BLK // _NB, quad, 0)
            return carry

        lax.fori_loop(0, cpt // _BLK, blk, 0)
        plsc.subcore_barrier()

        def wback(t, carry):
            off = sid * rpt + t * _CH
            pltpu.sync_copy(acc.at[pl.ds(off, _CH)],
                            out_hbm.at[pl.ds(cid * _NP + off, _CH)])
            return carry

        lax.fori_loop(0, rpt // _CH, wback, 0)

    return k(y, src2d, dst2d)


def _tc_layer1(xp, wlT, wrT, b):
    def body(x_ref, wl_ref, wr_ref, b_ref, y_ref, z_ref):
        xb = x_ref[...]
        y = jnp.dot(xb, wl_ref[...], preferred_element_type=jnp.float32)
        col = lax.broadcasted_iota(jnp.int32, (_R, _HP), 1)
        y_ref[...] = jnp.where(col == _H, 1.0, y)
        z_ref[...] = jnp.dot(xb, wr_ref[...],
                             preferred_element_type=jnp.float32) + b_ref[...]

    return pl.pallas_call(
        body,
        grid=(_NP // _R,),
        in_specs=[
            pl.BlockSpec((_R, _D), lambda i: (i, 0)),
            pl.BlockSpec((_D, _HP), lambda i: (0, 0)),
            pl.BlockSpec((_D, _HP), lambda i: (0, 0)),
            pl.BlockSpec((1, _HP), lambda i: (0, 0)),
        ],
        out_specs=[
            pl.BlockSpec((_R, _HP), lambda i: (i, 0)),
            pl.BlockSpec((_R, _HP), lambda i: (i, 0)),
        ],
        out_shape=[
            jax.ShapeDtypeStruct((_NP, _HP), jnp.float32),
            jax.ShapeDtypeStruct((_NP, _HP), jnp.float32),
        ],
    )(xp, wlT, wrT, b)


def _combine(sp_ref, z_ref):
    s = sp_ref[0] + sp_ref[1]
    col = lax.broadcasted_iota(jnp.int32, (_R, _HP), 1)
    deg = jnp.sum(jnp.where(col == _H, s, 0.0), axis=1, keepdims=True)
    d = jnp.maximum(deg, 1.0)
    return jnp.maximum(s / d + z_ref[...], 0.0)


def _tc_mid(spart, z, wlT, wrT, b):
    def body(sp_ref, z_ref, wl_ref, wr_ref, b_ref, y_ref, zn_ref):
        h = _combine(sp_ref, z_ref)
        y = jnp.dot(h, wl_ref[...], preferred_element_type=jnp.float32)
        col = lax.broadcasted_iota(jnp.int32, (_R, _HP), 1)
        y_ref[...] = jnp.where(col == _H, 1.0, y)
        zn_ref[...] = jnp.dot(h, wr_ref[...],
                              preferred_element_type=jnp.float32) + b_ref[...]

    return pl.pallas_call(
        body,
        grid=(_NP // _R,),
        in_specs=[
            pl.BlockSpec((2, _R, _HP), lambda i: (0, i, 0)),
            pl.BlockSpec((_R, _HP), lambda i: (i, 0)),
            pl.BlockSpec((_HP, _HP), lambda i: (0, 0)),
            pl.BlockSpec((_HP, _HP), lambda i: (0, 0)),
            pl.BlockSpec((1, _HP), lambda i: (0, 0)),
        ],
        out_specs=[
            pl.BlockSpec((_R, _HP), lambda i: (i, 0)),
            pl.BlockSpec((_R, _HP), lambda i: (i, 0)),
        ],
        out_shape=[
            jax.ShapeDtypeStruct((_NP, _HP), jnp.float32),
            jax.ShapeDtypeStruct((_NP, _HP), jnp.float32),
        ],
    )(spart, z, wlT, wrT, b)


def _tc_final(spart, z, batch2d, wlinT, blin2d):
    nsteps = _NP // _R

    def body(sp_ref, z_ref, batch_ref, wlin_ref, blin_ref, out_ref, pooled):
        i = pl.program_id(0)
        h = _combine(sp_ref, z_ref)
        bb = batch_ref[...]
        gid = lax.broadcasted_iota(jnp.int32, (_G, _R), 0)
        mask = (bb == gid).astype(jnp.float32)
        part = jnp.dot(mask, h, preferred_element_type=jnp.float32)

        @pl.when(i == 0)
        def _():
            pooled[...] = jnp.zeros((_G, _HP), jnp.float32)

        pooled[...] += part

        @pl.when(i == nsteps - 1)
        def _():
            out_ref[...] = jnp.dot(
                pooled[...], wlin_ref[...],
                preferred_element_type=jnp.float32) + blin_ref[...]

    return pl.pallas_call(
        body,
        grid=(nsteps,),
        in_specs=[
            pl.BlockSpec((2, _R, _HP), lambda i: (0, i, 0)),
            pl.BlockSpec((_R, _HP), lambda i: (i, 0)),
            pl.BlockSpec((1, _R), lambda i: (0, i)),
            pl.BlockSpec((_HP, _OUT), lambda i: (0, 0)),
            pl.BlockSpec((1, _OUT), lambda i: (0, 0)),
        ],
        out_specs=pl.BlockSpec((_G, _OUT), lambda i: (0, 0)),
        out_shape=jax.ShapeDtypeStruct((_G, _OUT), jnp.float32),
        scratch_shapes=[pltpu.VMEM((_G, _HP), jnp.float32)],
    )(spart, z, batch2d, wlinT, blin2d)


def kernel(x, edge_index, batch, W1l, b1l, W1r, W2l, b2l, W2r, W3l, b3l, W3r,
           Wlin, blin):
    e = edge_index.shape[1]
    cpt = -(-e // (32 * _CH))
    cpt = -(-cpt // _BLK) * _BLK
    ep = 32 * cpt * _CH

    xp = jnp.pad(x, ((0, _NP - _N), (0, 0)))
    pad_e = ep - e
    srcp = jnp.concatenate(
        [edge_index[0], jnp.full((pad_e,), _N, jnp.int32)]).reshape(
            32 * cpt, _CH)
    dstp = jnp.concatenate(
        [edge_index[1], jnp.full((pad_e,), _N, jnp.int32)]).reshape(
            32 * cpt, _CH)
    batchp = jnp.pad(batch, (0, _NP - _N),
                     constant_values=_G).reshape(1, _NP)

    pw = _HP - _H
    w1lT = jnp.pad(W1l.T, ((0, 0), (0, pw)))
    w1rT = jnp.pad(W1r.T, ((0, 0), (0, pw)))
    b1p = jnp.pad(b1l, (0, pw)).reshape(1, _HP)
    w2lT = jnp.pad(W2l.T, ((0, pw), (0, pw)))
    w2rT = jnp.pad(W2r.T, ((0, pw), (0, pw)))
    b2p = jnp.pad(b2l, (0, pw)).reshape(1, _HP)
    w3lT = jnp.pad(W3l.T, ((0, pw), (0, pw)))
    w3rT = jnp.pad(W3r.T, ((0, pw), (0, pw)))
    b3p = jnp.pad(b3l, (0, pw)).reshape(1, _HP)
    wlinT = jnp.pad(Wlin.T, ((0, pw), (0, 0)))
    blin2d = blin.reshape(1, _OUT)

    y1, z1 = _tc_layer1(xp, w1lT, w1rT, b1p)
    s1 = _seg_sum_sc(y1, srcp, dstp).reshape(2, _NP, _HP)
    y2, z2 = _tc_mid(s1, z1, w2lT, w2rT, b2p)
    s2 = _seg_sum_sc(y2, srcp, dstp).reshape(2, _NP, _HP)
    y3, z3 = _tc_mid(s2, z2, w3lT, w3rT, b3p)
    s3 = _seg_sum_sc(y3, srcp, dstp).reshape(2, _NP, _HP)
    return _tc_final(s3, z3, batchp, wlinT, blin2d)


# speedup vs baseline: 1.1080x; 1.1080x over previous
import functools

import jax
import jax.numpy as jnp
from jax import lax
from jax.experimental import pallas as pl
from jax.experimental.pallas import tpu as pltpu
from jax.experimental.pallas import tpu_sc as plsc

_N = 10000
_D = 128
_H = 88
_G = 64
_OUT = 100

_NP = 10240
_HP = 128
_CH = 64
_BLK = 8
_NB = 4
_R = 512


def _seg_sum_sc(y, src2d, dst2d):
    cpt = src2d.shape[0] // 32
    rpt = _NP // 16
    mesh = plsc.VectorSubcoreMesh(core_axis_name="c", subcore_axis_name="s")

    @functools.partial(
        pl.kernel,
        out_type=jax.ShapeDtypeStruct((2 * _NP, _HP), jnp.float32),
        mesh=mesh,
        scratch_types=[
            pltpu.VMEM((_BLK, _CH), jnp.int32),
            pltpu.VMEM((_BLK, _CH), jnp.int32),
            pltpu.VMEM((_NB, _CH, _HP), jnp.float32),
            pltpu.VMEM_SHARED((_NP, _HP), jnp.float32),
            pltpu.SemaphoreType.DMA,
            pltpu.SemaphoreType.DMA,
            pltpu.SemaphoreType.DMA,
            pltpu.SemaphoreType.DMA,
        ],
    )
    def k(y_hbm, src_hbm, dst_hbm, out_hbm, srcb, dstb, rows, acc,
          sem0, sem1, sem2, sem3):
        cid = lax.axis_index("c")
        sid = lax.axis_index("s")
        sems = (sem0, sem1, sem2, sem3)

        def zfill(i, carry):
            for kk in range(_HP // 16):
                rows[0, i, pl.ds(kk * 16, 16)] = jnp.zeros((16,), jnp.float32)
            return carry

        lax.fori_loop(0, _CH, zfill, 0)

        def zcopy(t, carry):
            pltpu.sync_copy(rows.at[0],
                            acc.at[pl.ds(sid * rpt + t * _CH, _CH)])
            return carry

        lax.fori_loop(0, rpt // _CH, zcopy, 0)
        plsc.subcore_barrier()

        def gcopy(jj, b):
            return pltpu.make_async_copy(y_hbm.at[srcb.at[jj]], rows.at[b],
                                         sems[b])

        def blk(bi, carry):
            base = (cid * 16 + sid) * cpt + bi * _BLK
            pltpu.sync_copy(src_hbm.at[pl.ds(base, _BLK)], srcb)
            pltpu.sync_copy(dst_hbm.at[pl.ds(base, _BLK)], dstb)
            for b in range(_NB):
                gcopy(b, b).start()

            def quad(p, c2):
                for b in range(_NB):
                    jj = p * _NB + b
                    gcopy(jj, b).wait()
                    pltpu.sync_copy(rows.at[b], acc.at[dstb.at[jj]],
                                    add=True)

                    @pl.when(jj + _NB < _BLK)
                    def _():
                        gcopy(jj + _NB, b).start()
                return c2

            lax.fori_loop(0, _BLK // _NB, quad, 0)
            return carry

        lax.fori_loop(0, cpt // _BLK, blk, 0)
        plsc.subcore_barrier()

        def wback(t, carry):
            off = sid * rpt + t * _CH
            pltpu.sync_copy(acc.at[pl.ds(off, _CH)],
                            out_hbm.at[pl.ds(cid * _NP + off, _CH)])
            return carry

        lax.fori_loop(0, rpt // _CH, wback, 0)

    return k(y, src2d, dst2d)


def _tc_layer1(xp, wlT, wrT, b):
    def body(x_ref, wl_ref, wr_ref, b_ref, y_ref, z_ref):
        xb = x_ref[...]
        y = jnp.dot(xb, wl_ref[...], preferred_element_type=jnp.float32)
        col = lax.broadcasted_iota(jnp.int32, (_R, _HP), 1)
        y_ref[...] = jnp.where(col == _H, 1.0, y)
        z_ref[...] = jnp.dot(xb, wr_ref[...],
                             preferred_element_type=jnp.float32) + b_ref[...]

    return pl.pallas_call(
        body,
        grid=(_NP // _R,),
        in_specs=[
            pl.BlockSpec((_R, _D), lambda i: (i, 0)),
            pl.BlockSpec((_D, _HP), lambda i: (0, 0)),
            pl.BlockSpec((_D, _HP), lambda i: (0, 0)),
            pl.BlockSpec((1, _HP), lambda i: (0, 0)),
        ],
        out_specs=[
            pl.BlockSpec((_R, _HP), lambda i: (i, 0)),
            pl.BlockSpec((_R, _HP), lambda i: (i, 0)),
        ],
        out_shape=[
            jax.ShapeDtypeStruct((_NP, _HP), jnp.float32),
            jax.ShapeDtypeStruct((_NP, _HP), jnp.float32),
        ],
    )(xp, wlT, wrT, b)


def _combine(sp_ref, z_ref):
    s = sp_ref[0] + sp_ref[1]
    col = lax.broadcasted_iota(jnp.int32, (_R, _HP), 1)
    deg = jnp.sum(jnp.where(col == _H, s, 0.0), axis=1, keepdims=True)
    d = jnp.maximum(deg, 1.0)
    return jnp.maximum(s / d + z_ref[...], 0.0)


def _tc_mid(spart, z, wlT, wrT, b):
    def body(sp_ref, z_ref, wl_ref, wr_ref, b_ref, y_ref, zn_ref):
        h = _combine(sp_ref, z_ref)
        y = jnp.dot(h, wl_ref[...], preferred_element_type=jnp.float32)
        col = lax.broadcasted_iota(jnp.int32, (_R, _HP), 1)
        y_ref[...] = jnp.where(col == _H, 1.0, y)
        zn_ref[...] = jnp.dot(h, wr_ref[...],
                              preferred_element_type=jnp.float32) + b_ref[...]

    return pl.pallas_call(
        body,
        grid=(_NP // _R,),
        in_specs=[
            pl.BlockSpec((2, _R, _HP), lambda i: (0, i, 0)),
            pl.BlockSpec((_R, _HP), lambda i: (i, 0)),
            pl.BlockSpec((_HP, _HP), lambda i: (0, 0)),
            pl.BlockSpec((_HP, _HP), lambda i: (0, 0)),
            pl.BlockSpec((1, _HP), lambda i: (0, 0)),
        ],
        out_specs=[
            pl.BlockSpec((_R, _HP), lambda i: (i, 0)),
            pl.BlockSpec((_R, _HP), lambda i: (i, 0)),
        ],
        out_shape=[
            jax.ShapeDtypeStruct((_NP, _HP), jnp.float32),
            jax.ShapeDtypeStruct((_NP, _HP), jnp.float32),
        ],
    )(spart, z, wlT, wrT, b)


def _tc_final(spart, z, batch2d, wlinT, blin2d):
    nsteps = _NP // _R

    def body(sp_ref, z_ref, batch_ref, wlin_ref, blin_ref, out_ref, pooled):
        i = pl.program_id(0)
        h = _combine(sp_ref, z_ref)
        bb = batch_ref[...]
        gid = lax.broadcasted_iota(jnp.int32, (_G, _R), 0)
        mask = (bb == gid).astype(jnp.float32)
        part = jnp.dot(mask, h, preferred_element_type=jnp.float32)

        @pl.when(i == 0)
        def _():
            pooled[...] = jnp.zeros((_G, _HP), jnp.float32)

        pooled[...] += part

        @pl.when(i == nsteps - 1)
        def _():
            out_ref[...] = jnp.dot(
                pooled[...], wlin_ref[...],
                preferred_element_type=jnp.float32) + blin_ref[...]

    return pl.pallas_call(
        body,
        grid=(nsteps,),
        in_specs=[
            pl.BlockSpec((2, _R, _HP), lambda i: (0, i, 0)),
            pl.BlockSpec((_R, _HP), lambda i: (i, 0)),
            pl.BlockSpec((1, _R), lambda i: (0, i)),
            pl.BlockSpec((_HP, _OUT), lambda i: (0, 0)),
            pl.BlockSpec((1, _OUT), lambda i: (0, 0)),
        ],
        out_specs=pl.BlockSpec((_G, _OUT), lambda i: (0, 0)),
        out_shape=jax.ShapeDtypeStruct((_G, _OUT), jnp.float32),
        scratch_shapes=[pltpu.VMEM((_G, _HP), jnp.float32)],
    )(spart, z, batch2d, wlinT, blin2d)


def kernel(x, edge_index, batch, W1l, b1l, W1r, W2l, b2l, W2r, W3l, b3l, W3r,
           Wlin, blin):
    e = edge_index.shape[1]
    cpt = -(-e // (32 * _CH))
    cpt = -(-cpt // _BLK) * _BLK
    ep = 32 * cpt * _CH

    xp = jnp.pad(x, ((0, _NP - _N), (0, 0)))
    pad_e = ep - e
    srcp = jnp.concatenate(
        [edge_index[0], jnp.full((pad_e,), _N, jnp.int32)]).reshape(
            32 * cpt, _CH)
    dstp = jnp.concatenate(
        [edge_index[1], jnp.full((pad_e,), _N, jnp.int32)]).reshape(
            32 * cpt, _CH)
    batchp = jnp.pad(batch, (0, _NP - _N),
                     constant_values=_G).reshape(1, _NP)

    pw = _HP - _H
    w1lT = jnp.pad(W1l.T, ((0, 0), (0, pw)))
    w1rT = jnp.pad(W1r.T, ((0, 0), (0, pw)))
    b1p = jnp.pad(b1l, (0, pw)).reshape(1, _HP)
    w2lT = jnp.pad(W2l.T, ((0, pw), (0, pw)))
    w2rT = jnp.pad(W2r.T, ((0, pw), (0, pw)))
    b2p = jnp.pad(b2l, (0, pw)).reshape(1, _HP)
    w3lT = jnp.pad(W3l.T, ((0, pw), (0, pw)))
    w3rT = jnp.pad(W3r.T, ((0, pw), (0, pw)))
    b3p = jnp.pad(b3l, (0, pw)).reshape(1, _HP)
    wlinT = jnp.pad(Wlin.T, ((0, pw), (0, 0)))
    blin2d = blin.reshape(1, _OUT)

    y1, z1 = _tc_layer1(xp, w1lT, w1rT, b1p)
    s1 = _seg_sum_sc(y1, srcp, dstp).reshape(2, _NP, _HP)
    y2, z2 = _tc_mid(s1, z1, w2lT, w2rT, b2p)
    s2 = _seg_sum_sc(y2, srcp, dstp).reshape(2, _NP, _HP)
    y3, z3 = _tc_mid(s2, z2, w3lT, w3rT, b3p)
    s3 = _seg_sum_sc(y3, srcp, dstp).reshape(2, _NP, _HP)
    return _tc_final(s3, z3, batchp, wlinT, blin2d)
